# Initial kernel scaffold; baseline (speedup 1.0000x reference)
import jax
import jax.numpy as jnp
from jax.experimental import pallas as pl


def kernel(pred):
    raise NotImplementedError("write your pallas kernel here")


# speedup vs baseline: 208.6809x; 208.6809x over previous
import functools

import jax
import jax.numpy as jnp
from jax import lax
from jax.experimental import pallas as pl
from jax.experimental.pallas import tpu as pltpu
from jax.experimental.pallas import tpu_sc as plsc

H = 384
W = 384
N = 16
NW = 32
CR = 48
NCHUNK = 4
NCB = W // 16
F = (H - 1) * (W - 1) * 2
SCALE = 1.0 / (N * F)
BUFLEN = (CR + 2) * W

_mesh = plsc.VectorSubcoreMesh(core_axis_name="c", subcore_axis_name="s")


@functools.partial(
    pl.kernel,
    out_type=jax.ShapeDtypeStruct((NW, 16), jnp.float32),
    mesh=_mesh,
    scratch_types=[
        pltpu.VMEM((BUFLEN,), jnp.float32),
        pltpu.VMEM((BUFLEN,), jnp.float32),
        pltpu.VMEM((16,), jnp.float32),
    ],
)
def _det_partials(pred_hbm, out_hbm, s_buf, t_buf, acc_v):
    wid = lax.axis_index("s") * 2 + lax.axis_index("c")
    n = wid // 2
    h = wid % 2

    zero16 = jnp.zeros((16,), jnp.float32)
    lane = lax.iota(jnp.int32, 16)
    ones = jnp.ones((16,), jnp.float32)
    mask_first = jnp.where(lane >= 1, 1.0, 0.0).astype(jnp.float32)
    mask_last = jnp.where(lane < 14, 1.0, 0.0).astype(jnp.float32)

    for i in range(NCB):
        s_buf[pl.ds((CR + 1) * W + 16 * i, 16)] = zero16
        t_buf[pl.ds((CR + 1) * W + 16 * i, 16)] = zero16

    acc = zero16
    for j in range(NCHUNK):
        d0 = h * 192 + j * CR
        s_in = jnp.minimum(d0, H - 1 - CR)
        lo = jnp.maximum(d0, 1)

        pltpu.sync_copy(pred_hbm.at[n, 0, pl.ds(s_in * W, (CR + 1) * W)],
                        s_buf.at[pl.ds(0, (CR + 1) * W)])
        pltpu.sync_copy(pred_hbm.at[n, 1, pl.ds(s_in * W, (CR + 1) * W)],
                        t_buf.at[pl.ds(0, (CR + 1) * W)])

        for cb in range(NCB):
            c0 = 16 * cb
            mf = mask_first if cb == 0 else (mask_last if cb == NCB - 1 else ones)

            s_a = s_buf[pl.ds(c0, 16)]
            s_b = s_buf[pl.ds(c0 + 1, 16)]
            t_a = t_buf[pl.ds(c0, 16)]
            t_b = t_buf[pl.ds(c0 + 1, 16)]
            dxs0 = s_b - s_a
            dxt0 = t_b - t_a

            def row_body(k, carry, c0=c0, mf=mf, s_in=s_in, lo=lo):
                s_a, s_b, t_a, t_b, dxs0, dxt0, acc = carry
                base = (k + 1) * W + c0
                s_a1 = s_buf[pl.ds(base, 16)]
                s_b1 = s_buf[pl.ds(base + 1, 16)]
                t_a1 = t_buf[pl.ds(base, 16)]
                t_b1 = t_buf[pl.ds(base + 1, 16)]
                dys = s_a1 - s_a
                dyt = t_a1 - t_a
                dxs1 = s_b1 - s_a1
                dxt1 = t_b1 - t_a1
                det1 = dxs0 * dyt - dxt0 * dys
                dys_b = s_b1 - s_b
                dyt_b = t_b1 - t_b
                det2 = dxs1 * dyt_b - dxt1 * dys_b
                r = s_in + k
                valid = jnp.logical_and(r >= lo, r <= H - 3)
                rv = jnp.where(valid, 1.0, 0.0).astype(jnp.float32)
                mm = mf * rv
                term = jnp.minimum(det1, 0.0) + jnp.minimum(det2, 0.0)
                acc = acc - mm * term
                return (s_a1, s_b1, t_a1, t_b1, dxs1, dxt1, acc)

            carry = (s_a, s_b, t_a, t_b, dxs0, dxt0, acc)
            carry = lax.fori_loop(0, CR, row_body, carry)
            acc = carry[6]

    acc_v[...] = acc
    pltpu.sync_copy(acc_v, out_hbm.at[wid])


def _reduce_body(p_ref, o_ref):
    o_ref[0, 0] = jnp.sum(p_ref[...]) * SCALE


_reduce = pl.pallas_call(
    _reduce_body,
    out_shape=jax.ShapeDtypeStruct((1, 1), jnp.float32),
    out_specs=pl.BlockSpec(memory_space=pltpu.SMEM),
)


def kernel(pred):
    pred2 = pred.reshape(N, 2, H * W)
    partials = _det_partials(pred2)
    return _reduce(partials)[0, 0]


# speedup vs baseline: 220.1903x; 1.0552x over previous
import functools

import jax
import jax.numpy as jnp
from jax import lax
from jax.experimental import pallas as pl
from jax.experimental.pallas import tpu as pltpu
from jax.experimental.pallas import tpu_sc as plsc

H = 384
W = 384
N = 16
NW = 32
CR = 48
NCHUNK = 4
NCB = W // 16
F = (H - 1) * (W - 1) * 2
SCALE = 1.0 / (N * F)
BUFLEN = (CR + 2) * W

_mesh = plsc.VectorSubcoreMesh(core_axis_name="c", subcore_axis_name="s")


@functools.partial(
    pl.kernel,
    out_type=jax.ShapeDtypeStruct((NW, 16), jnp.float32),
    mesh=_mesh,
    scratch_types=[
        pltpu.VMEM((BUFLEN,), jnp.float32),
        pltpu.VMEM((BUFLEN,), jnp.float32),
        pltpu.VMEM((16,), jnp.float32),
    ],
)
def _det_partials(pred_hbm, out_hbm, s_buf, t_buf, acc_v):
    wid = lax.axis_index("s") * 2 + lax.axis_index("c")
    n = wid // 2
    h = wid % 2

    zero16 = jnp.zeros((16,), jnp.float32)
    lane = lax.iota(jnp.int32, 16)
    ones = jnp.ones((16,), jnp.float32)
    mask_first = jnp.where(lane >= 1, 1.0, 0.0).astype(jnp.float32)
    mask_last = jnp.where(lane < 14, 1.0, 0.0).astype(jnp.float32)

    for i in range(NCB):
        s_buf[pl.ds((CR + 1) * W + 16 * i, 16)] = zero16
        t_buf[pl.ds((CR + 1) * W + 16 * i, 16)] = zero16

    def chunk_body(j, acc):
        d0 = h * 192 + j * CR
        s_in = jnp.minimum(d0, H - 1 - CR)
        lo = jnp.maximum(d0, 1)

        pltpu.sync_copy(pred_hbm.at[n, 0, pl.ds(s_in * W, (CR + 1) * W)],
                        s_buf.at[pl.ds(0, (CR + 1) * W)])
        pltpu.sync_copy(pred_hbm.at[n, 1, pl.ds(s_in * W, (CR + 1) * W)],
                        t_buf.at[pl.ds(0, (CR + 1) * W)])

        for cb in range(NCB):
            c0 = 16 * cb
            mf = mask_first if cb == 0 else (mask_last if cb == NCB - 1 else ones)

            s_a = s_buf[pl.ds(c0, 16)]
            s_b = s_buf[pl.ds(c0 + 1, 16)]
            t_a = t_buf[pl.ds(c0, 16)]
            t_b = t_buf[pl.ds(c0 + 1, 16)]
            dxs0 = s_b - s_a
            dxt0 = t_b - t_a

            def row_body(k, carry, c0=c0, mf=mf, s_in=s_in, lo=lo):
                s_a, s_b, t_a, t_b, dxs0, dxt0, acc = carry
                base = (k + 1) * W + c0
                s_a1 = s_buf[pl.ds(base, 16)]
                s_b1 = s_buf[pl.ds(base + 1, 16)]
                t_a1 = t_buf[pl.ds(base, 16)]
                t_b1 = t_buf[pl.ds(base + 1, 16)]
                dys = s_a1 - s_a
                dyt = t_a1 - t_a
                dxs1 = s_b1 - s_a1
                dxt1 = t_b1 - t_a1
                det1 = dxs0 * dyt - dxt0 * dys
                dys_b = s_b1 - s_b
                dyt_b = t_b1 - t_b
                det2 = dxs1 * dyt_b - dxt1 * dys_b
                r = s_in + k
                valid = jnp.logical_and(r >= lo, r <= H - 3)
                rv = jnp.where(valid, 1.0, 0.0).astype(jnp.float32)
                mm = mf * rv
                term = jnp.minimum(det1, 0.0) + jnp.minimum(det2, 0.0)
                acc = acc - mm * term
                return (s_a1, s_b1, t_a1, t_b1, dxs1, dxt1, acc)

            carry = (s_a, s_b, t_a, t_b, dxs0, dxt0, acc)
            carry = plsc.parallel_loop(0, CR, unroll=4, carry=carry)(row_body)
            acc = carry[6]
        return acc

    acc = lax.fori_loop(0, NCHUNK, chunk_body, zero16)

    acc_v[...] = acc
    pltpu.sync_copy(acc_v, out_hbm.at[wid])


def _reduce_body(p_ref, o_ref):
    o_ref[0, 0] = jnp.sum(p_ref[...]) * SCALE


_reduce = pl.pallas_call(
    _reduce_body,
    out_shape=jax.ShapeDtypeStruct((1, 1), jnp.float32),
    out_specs=pl.BlockSpec(memory_space=pltpu.SMEM),
)


def kernel(pred):
    pred2 = pred.reshape(N, 2, H * W)
    partials = _det_partials(pred2)
    return _reduce(partials)[0, 0]


# speedup vs baseline: 277.3133x; 1.2594x over previous
import functools

import jax
import jax.numpy as jnp
from jax import lax
from jax.experimental import pallas as pl
from jax.experimental.pallas import tpu as pltpu
from jax.experimental.pallas import tpu_sc as plsc

H = 384
W = 384
N = 16
NW = 32
CR = 48
CRW = 56
CRL = 55
NCHUNK = 4
NCB = W // 16
WPAD = W + 16
F = (H - 1) * (W - 1) * 2
SCALE = 1.0 / (N * F)

_mesh = plsc.VectorSubcoreMesh(core_axis_name="c", subcore_axis_name="s")


@functools.partial(
    pl.kernel,
    out_type=jax.ShapeDtypeStruct((NW, 16), jnp.float32),
    mesh=_mesh,
    scratch_types=[
        pltpu.VMEM((CRW, WPAD), jnp.float32),
        pltpu.VMEM((CRW, WPAD), jnp.float32),
        pltpu.VMEM((16,), jnp.float32),
    ],
)
def _det_partials(pred_hbm, out_hbm, s_buf, t_buf, acc_v):
    wid = lax.axis_index("s") * 2 + lax.axis_index("c")
    n = wid // 2
    h = wid % 2

    zero16 = jnp.zeros((16,), jnp.float32)
    lane = lax.iota(jnp.int32, 16)
    ones = jnp.ones((16,), jnp.float32)
    mask_first = jnp.where(lane >= 1, 1.0, 0.0).astype(jnp.float32)
    mask_last = jnp.where(lane < 14, 1.0, 0.0).astype(jnp.float32)

    for i in range(CRW):
        s_buf[i, pl.ds(W, 16)] = zero16
        t_buf[i, pl.ds(W, 16)] = zero16

    def chunk_body(j, acc):
        d0 = h * 192 + j * CR
        s_in = pl.multiple_of(jnp.minimum(d0, H - CRW), 8)
        lo = jnp.maximum(d0, 1)
        hi = jnp.minimum(d0 + CR - 1, H - 3)

        pltpu.sync_copy(pred_hbm.at[n, 0, pl.ds(s_in, CRW), :],
                        s_buf.at[pl.ds(0, CRW), pl.ds(0, W)])
        pltpu.sync_copy(pred_hbm.at[n, 1, pl.ds(s_in, CRW), :],
                        t_buf.at[pl.ds(0, CRW), pl.ds(0, W)])

        for cb in range(NCB):
            c0 = 16 * cb
            mf = mask_first if cb == 0 else (mask_last if cb == NCB - 1 else ones)

            s_a = s_buf[0, pl.ds(c0, 16)]
            s_b = s_buf[0, pl.ds(c0 + 1, 16)]
            t_a = t_buf[0, pl.ds(c0, 16)]
            t_b = t_buf[0, pl.ds(c0 + 1, 16)]
            dxs0 = s_b - s_a
            dxt0 = t_b - t_a

            def row_body(k, carry, c0=c0, mf=mf, s_in=s_in, lo=lo, hi=hi):
                s_a, s_b, t_a, t_b, dxs0, dxt0, acc = carry
                s_a1 = s_buf[k + 1, pl.ds(c0, 16)]
                s_b1 = s_buf[k + 1, pl.ds(c0 + 1, 16)]
                t_a1 = t_buf[k + 1, pl.ds(c0, 16)]
                t_b1 = t_buf[k + 1, pl.ds(c0 + 1, 16)]
                dys = s_a1 - s_a
                dyt = t_a1 - t_a
                dxs1 = s_b1 - s_a1
                dxt1 = t_b1 - t_a1
                det1 = dxs0 * dyt - dxt0 * dys
                dys_b = s_b1 - s_b
                dyt_b = t_b1 - t_b
                det2 = dxs1 * dyt_b - dxt1 * dys_b
                r = s_in + k
                valid = jnp.logical_and(r >= lo, r <= hi)
                rv = jnp.where(valid, 1.0, 0.0).astype(jnp.float32)
                mm = mf * rv
                term = jnp.minimum(det1, 0.0) + jnp.minimum(det2, 0.0)
                acc = acc - mm * term
                return (s_a1, s_b1, t_a1, t_b1, dxs1, dxt1, acc)

            carry = (s_a, s_b, t_a, t_b, dxs0, dxt0, acc)
            carry = plsc.parallel_loop(0, CRL, unroll=5, carry=carry)(row_body)
            acc = carry[6]
        return acc

    acc = lax.fori_loop(0, NCHUNK, chunk_body, zero16)

    acc_v[...] = acc
    pltpu.sync_copy(acc_v, out_hbm.at[wid])


def _reduce_body(p_ref, o_ref):
    o_ref[0, 0] = jnp.sum(p_ref[...]) * SCALE


_reduce = pl.pallas_call(
    _reduce_body,
    out_shape=jax.ShapeDtypeStruct((1, 1), jnp.float32),
    out_specs=pl.BlockSpec(memory_space=pltpu.SMEM),
)


def kernel(pred):
    partials = _det_partials(pred)
    return _reduce(partials)[0, 0]


# speedup vs baseline: 300.0716x; 1.0821x over previous
import functools

import jax
import jax.numpy as jnp
from jax import lax
from jax.experimental import pallas as pl
from jax.experimental.pallas import tpu as pltpu
from jax.experimental.pallas import tpu_sc as plsc

H = 384
W = 384
N = 16
NW = 32
CR = 48
CRW = 56
CRL = 55
NCHUNK = 4
NCB = W // 16
WPAD = W + 16
F = (H - 1) * (W - 1) * 2
SCALE = 1.0 / (N * F)

_mesh = plsc.VectorSubcoreMesh(core_axis_name="c", subcore_axis_name="s")


@functools.partial(
    pl.kernel,
    out_type=jax.ShapeDtypeStruct((NW, 16), jnp.float32),
    mesh=_mesh,
    scratch_types=[
        pltpu.VMEM((CRW, W), jnp.float32),
        pltpu.VMEM((CRW, W), jnp.float32),
        pltpu.VMEM((16,), jnp.float32),
    ],
)
def _det_partials(pred_hbm, out_hbm, s_buf, t_buf, acc_v):
    wid = lax.axis_index("s") * 2 + lax.axis_index("c")
    n = wid // 2
    h = wid % 2

    zero16 = jnp.zeros((16,), jnp.float32)
    lane = lax.iota(jnp.int32, 16)
    ones = jnp.ones((16,), jnp.float32)
    mask_first = jnp.where(lane >= 1, 1.0, 0.0).astype(jnp.float32)
    mask_last = jnp.where(lane < 14, 1.0, 0.0).astype(jnp.float32)
    perm = jnp.minimum(lane + 1, 15)

    def chunk_body(j, acc):
        d0 = h * 192 + j * CR
        s_in = pl.multiple_of(jnp.minimum(d0, H - CRW), 8)
        lo = jnp.maximum(d0, 1)
        hi = jnp.minimum(d0 + CR - 1, H - 3)

        pltpu.sync_copy(pred_hbm.at[n, 0, pl.ds(s_in, CRW), :], s_buf)
        pltpu.sync_copy(pred_hbm.at[n, 1, pl.ds(s_in, CRW), :], t_buf)

        def shifted_from(buf, row, cb, aligned):
            if cb < NCB - 1:
                return buf[row, pl.ds(16 * cb + 1, 16)]
            return aligned.at[perm].get(mode="promise_in_bounds")

        for cb in range(NCB):
            c0 = 16 * cb
            mf = mask_first if cb == 0 else (mask_last if cb == NCB - 1 else ones)

            s_a = s_buf[0, pl.ds(c0, 16)]
            s_b = shifted_from(s_buf, 0, cb, s_a)
            t_a = t_buf[0, pl.ds(c0, 16)]
            t_b = shifted_from(t_buf, 0, cb, t_a)
            dxs0 = s_b - s_a
            dxt0 = t_b - t_a

            def row_body(k, carry, c0=c0, cb=cb, mf=mf, s_in=s_in, lo=lo, hi=hi):
                s_a, s_b, t_a, t_b, dxs0, dxt0, acc = carry
                s_a1 = s_buf[k + 1, pl.ds(c0, 16)]
                s_b1 = shifted_from(s_buf, k + 1, cb, s_a1)
                t_a1 = t_buf[k + 1, pl.ds(c0, 16)]
                t_b1 = shifted_from(t_buf, k + 1, cb, t_a1)
                dys = s_a1 - s_a
                dyt = t_a1 - t_a
                dxs1 = s_b1 - s_a1
                dxt1 = t_b1 - t_a1
                det1 = dxs0 * dyt - dxt0 * dys
                dys_b = s_b1 - s_b
                dyt_b = t_b1 - t_b
                det2 = dxs1 * dyt_b - dxt1 * dys_b
                r = s_in + k
                valid = jnp.logical_and(r >= lo, r <= hi)
                rv = jnp.where(valid, 1.0, 0.0).astype(jnp.float32)
                mm = mf * rv
                term = jnp.minimum(det1, 0.0) + jnp.minimum(det2, 0.0)
                acc = acc - mm * term
                return (s_a1, s_b1, t_a1, t_b1, dxs1, dxt1, acc)

            carry = (s_a, s_b, t_a, t_b, dxs0, dxt0, acc)
            carry = plsc.parallel_loop(0, CRL, unroll=5, carry=carry)(row_body)
            acc = carry[6]
        return acc

    acc = lax.fori_loop(0, NCHUNK, chunk_body, zero16)

    acc_v[...] = acc
    pltpu.sync_copy(acc_v, out_hbm.at[wid])


def _reduce_body(p_ref, o_ref):
    o_ref[0, 0] = jnp.sum(p_ref[...]) * SCALE


_reduce = pl.pallas_call(
    _reduce_body,
    out_shape=jax.ShapeDtypeStruct((1, 1), jnp.float32),
    out_specs=pl.BlockSpec(memory_space=pltpu.SMEM),
)


def kernel(pred):
    partials = _det_partials(pred)
    return _reduce(partials)[0, 0]


# speedup vs baseline: 321.1042x; 1.0701x over previous
import functools

import jax
import jax.numpy as jnp
from jax import lax
from jax.experimental import pallas as pl
from jax.experimental.pallas import tpu as pltpu
from jax.experimental.pallas import tpu_sc as plsc

H = 384
W = 384
N = 16
NW = 32
CR = 48
NCHUNK = 4
NCB = W // 16
F = (H - 1) * (W - 1) * 2
SCALE = 1.0 / (N * F)

_mesh = plsc.VectorSubcoreMesh(core_axis_name="c", subcore_axis_name="s")


@functools.partial(
    pl.kernel,
    out_type=jax.ShapeDtypeStruct((NW, 16), jnp.float32),
    mesh=_mesh,
    scratch_types=[
        pltpu.VMEM((2, CR, W), jnp.float32),
        pltpu.VMEM((2, CR, W), jnp.float32),
        pltpu.VMEM((W,), jnp.float32),
        pltpu.VMEM((W,), jnp.float32),
        pltpu.VMEM((8, W), jnp.float32),
        pltpu.VMEM((16,), jnp.float32),
        pltpu.SemaphoreType.DMA((2,)),
        pltpu.SemaphoreType.DMA((2,)),
    ],
)
def _det_partials(pred_hbm, out_hbm, s_sl, t_sl, s_halo, t_halo, stage, acc_v,
                  sem_s, sem_t):
    wid = lax.axis_index("s") * 2 + lax.axis_index("c")
    n = wid // 2
    h = wid % 2

    zero16 = jnp.zeros((16,), jnp.float32)
    lane = lax.iota(jnp.int32, 16)
    ones = jnp.ones((16,), jnp.float32)
    mask_first = jnp.where(lane >= 1, 1.0, 0.0).astype(jnp.float32)
    mask_last = jnp.where(lane < 14, 1.0, 0.0).astype(jnp.float32)
    perm = jnp.minimum(lane + 1, 15)

    base = h * 192
    lo = jnp.maximum(base - 1, 1)
    hi = jnp.minimum(base + 190, H - 3)

    def win_src(ch_ref_idx, j):
        e = pl.multiple_of(base + j * CR, 8)
        return pred_hbm.at[n, ch_ref_idx, pl.ds(e, CR), :]

    pltpu.async_copy(win_src(0, 0), s_sl.at[0], sem_s.at[0])
    pltpu.async_copy(win_src(1, 0), t_sl.at[0], sem_t.at[0])

    @pl.when(h == 0)
    def _():
        for i in range(NCB):
            s_halo[pl.ds(16 * i, 16)] = zero16
            t_halo[pl.ds(16 * i, 16)] = zero16

    @pl.when(h == 1)
    def _():
        pltpu.sync_copy(pred_hbm.at[n, 0, pl.ds(184, 8), :], stage)
        for i in range(NCB):
            s_halo[pl.ds(16 * i, 16)] = stage[7, pl.ds(16 * i, 16)]
        pltpu.sync_copy(pred_hbm.at[n, 1, pl.ds(184, 8), :], stage)
        for i in range(NCB):
            t_halo[pl.ds(16 * i, 16)] = stage[7, pl.ds(16 * i, 16)]

    def chunk_body(j, acc):
        slot = j % 2
        other = 1 - slot

        pltpu.make_async_copy(win_src(0, j), s_sl.at[slot], sem_s.at[slot]).wait()
        pltpu.make_async_copy(win_src(1, j), t_sl.at[slot], sem_t.at[slot]).wait()

        @pl.when(j > 0)
        def _():
            for i in range(NCB):
                s_halo[pl.ds(16 * i, 16)] = s_sl[other, CR - 1, pl.ds(16 * i, 16)]
                t_halo[pl.ds(16 * i, 16)] = t_sl[other, CR - 1, pl.ds(16 * i, 16)]

        @pl.when(j < NCHUNK - 1)
        def _():
            pltpu.async_copy(win_src(0, j + 1), s_sl.at[other], sem_s.at[other])
            pltpu.async_copy(win_src(1, j + 1), t_sl.at[other], sem_t.at[other])

        r0 = base + j * CR - 1

        def shifted_from(buf, row, cb, aligned):
            if cb < NCB - 1:
                return buf[slot, row, pl.ds(16 * cb + 1, 16)]
            return aligned.at[perm].get(mode="promise_in_bounds")

        for cb in range(NCB):
            c0 = 16 * cb
            mf = mask_first if cb == 0 else (mask_last if cb == NCB - 1 else ones)

            s_a = s_halo[pl.ds(c0, 16)]
            t_a = t_halo[pl.ds(c0, 16)]
            if cb < NCB - 1:
                s_b = s_halo[pl.ds(c0 + 1, 16)]
                t_b = t_halo[pl.ds(c0 + 1, 16)]
            else:
                s_b = s_a.at[perm].get(mode="promise_in_bounds")
                t_b = t_a.at[perm].get(mode="promise_in_bounds")
            dxs0 = s_b - s_a
            dxt0 = t_b - t_a

            def row_body(k, carry, c0=c0, cb=cb, mf=mf, r0=r0):
                s_a, s_b, t_a, t_b, dxs0, dxt0, acc = carry
                s_a1 = s_sl[slot, k, pl.ds(c0, 16)]
                s_b1 = shifted_from(s_sl, k, cb, s_a1)
                t_a1 = t_sl[slot, k, pl.ds(c0, 16)]
                t_b1 = shifted_from(t_sl, k, cb, t_a1)
                dys = s_a1 - s_a
                dyt = t_a1 - t_a
                dxs1 = s_b1 - s_a1
                dxt1 = t_b1 - t_a1
                det1 = dxs0 * dyt - dxt0 * dys
                dys_b = s_b1 - s_b
                dyt_b = t_b1 - t_b
                det2 = dxs1 * dyt_b - dxt1 * dys_b
                r = r0 + k
                valid = jnp.logical_and(r >= lo, r <= hi)
                rv = jnp.where(valid, 1.0, 0.0).astype(jnp.float32)
                mm = mf * rv
                term = jnp.minimum(det1, 0.0) + jnp.minimum(det2, 0.0)
                acc = acc - mm * term
                return (s_a1, s_b1, t_a1, t_b1, dxs1, dxt1, acc)

            carry = (s_a, s_b, t_a, t_b, dxs0, dxt0, acc)
            carry = plsc.parallel_loop(0, CR, unroll=4, carry=carry)(row_body)
            acc = carry[6]
        return acc

    acc = lax.fori_loop(0, NCHUNK, chunk_body, jnp.zeros((16,), jnp.float32))

    acc_v[...] = acc
    pltpu.sync_copy(acc_v, out_hbm.at[wid])


def _reduce_body(p_ref, o_ref):
    o_ref[0, 0] = jnp.sum(p_ref[...]) * SCALE


_reduce = pl.pallas_call(
    _reduce_body,
    out_shape=jax.ShapeDtypeStruct((1, 1), jnp.float32),
    out_specs=pl.BlockSpec(memory_space=pltpu.SMEM),
)


def kernel(pred):
    partials = _det_partials(pred)
    return _reduce(partials)[0, 0]


# speedup vs baseline: 337.6766x; 1.0516x over previous
import functools

import jax
import jax.numpy as jnp
from jax import lax
from jax.experimental import pallas as pl
from jax.experimental.pallas import tpu as pltpu
from jax.experimental.pallas import tpu_sc as plsc

H = 384
W = 384
N = 16
NW = 32
CR = 48
NCHUNK = 4
NCB = W // 16
RU = 4
F = (H - 1) * (W - 1) * 2
SCALE = 1.0 / (N * F)

_mesh = plsc.VectorSubcoreMesh(core_axis_name="c", subcore_axis_name="s")


@functools.partial(
    pl.kernel,
    out_type=jax.ShapeDtypeStruct((NW, 16), jnp.float32),
    mesh=_mesh,
    scratch_types=[
        pltpu.VMEM((2, CR, W), jnp.float32),
        pltpu.VMEM((2, CR, W), jnp.float32),
        pltpu.VMEM((W,), jnp.float32),
        pltpu.VMEM((W,), jnp.float32),
        pltpu.VMEM((8, W), jnp.float32),
        pltpu.VMEM((16,), jnp.float32),
        pltpu.SemaphoreType.DMA((2,)),
        pltpu.SemaphoreType.DMA((2,)),
    ],
)
def _det_partials(pred_hbm, out_hbm, s_sl, t_sl, s_halo, t_halo, stage, acc_v,
                  sem_s, sem_t):
    wid = lax.axis_index("s") * 2 + lax.axis_index("c")
    n = wid // 2
    h = wid % 2

    lane = lax.iota(jnp.int32, 16)
    mask_first = jnp.where(lane >= 1, 1.0, 0.0).astype(jnp.float32)
    mask_last = jnp.where(lane < 14, 1.0, 0.0).astype(jnp.float32)
    perm = jnp.minimum(lane + 1, 15)

    base = h * 192
    edge_k = jnp.where(h == 0, 1, CR - 1)
    edge_j = jnp.where(h == 0, 0, NCHUNK - 1)

    def win_src(ch_ref_idx, j):
        e = pl.multiple_of(base + j * CR, 8)
        return pred_hbm.at[n, ch_ref_idx, pl.ds(e, CR), :]

    pltpu.async_copy(win_src(0, 0), s_sl.at[0], sem_s.at[0])
    pltpu.async_copy(win_src(1, 0), t_sl.at[0], sem_t.at[0])

    @pl.when(h == 1)
    def _():
        pltpu.sync_copy(pred_hbm.at[n, 0, pl.ds(184, 8), :], stage)
        for i in range(NCB):
            s_halo[pl.ds(16 * i, 16)] = stage[7, pl.ds(16 * i, 16)]
        pltpu.sync_copy(pred_hbm.at[n, 1, pl.ds(184, 8), :], stage)
        for i in range(NCB):
            t_halo[pl.ds(16 * i, 16)] = stage[7, pl.ds(16 * i, 16)]

    def chunk_body(j, acc):
        slot = j % 2
        other = 1 - slot

        pltpu.make_async_copy(win_src(0, j), s_sl.at[slot], sem_s.at[slot]).wait()
        pltpu.make_async_copy(win_src(1, j), t_sl.at[slot], sem_t.at[slot]).wait()

        @pl.when(jnp.logical_and(j == 0, h == 0))
        def _():
            for i in range(NCB):
                s_halo[pl.ds(16 * i, 16)] = s_sl[slot, 0, pl.ds(16 * i, 16)]
                t_halo[pl.ds(16 * i, 16)] = t_sl[slot, 0, pl.ds(16 * i, 16)]

        @pl.when(j > 0)
        def _():
            for i in range(NCB):
                s_halo[pl.ds(16 * i, 16)] = s_sl[other, CR - 1, pl.ds(16 * i, 16)]
                t_halo[pl.ds(16 * i, 16)] = t_sl[other, CR - 1, pl.ds(16 * i, 16)]

        @pl.when(j < NCHUNK - 1)
        def _():
            pltpu.async_copy(win_src(0, j + 1), s_sl.at[other], sem_s.at[other])
            pltpu.async_copy(win_src(1, j + 1), t_sl.at[other], sem_t.at[other])

        ef = jnp.where(j == edge_j, 1.0, 0.0).astype(jnp.float32)

        def loads(row, cb):
            c0 = 16 * cb
            s_a1 = s_sl[slot, row, pl.ds(c0, 16)]
            t_a1 = t_sl[slot, row, pl.ds(c0, 16)]
            if cb < NCB - 1:
                s_b1 = s_sl[slot, row, pl.ds(c0 + 1, 16)]
                t_b1 = t_sl[slot, row, pl.ds(c0 + 1, 16)]
            else:
                s_b1 = s_a1.at[perm].get(mode="promise_in_bounds")
                t_b1 = t_a1.at[perm].get(mode="promise_in_bounds")
            return s_a1, s_b1, t_a1, t_b1

        def term_of(prev, cur):
            s_a, s_b, t_a, t_b, dxs0, dxt0 = prev
            s_a1, s_b1, t_a1, t_b1 = cur
            dys = s_a1 - s_a
            dyt = t_a1 - t_a
            dxs1 = s_b1 - s_a1
            dxt1 = t_b1 - t_a1
            det1 = dxs0 * dyt - dxt0 * dys
            dys_b = s_b1 - s_b
            dyt_b = t_b1 - t_b
            det2 = dxs1 * dyt_b - dxt1 * dys_b
            term = jnp.minimum(det1, 0.0) + jnp.minimum(det2, 0.0)
            return term, (s_a1, s_b1, t_a1, t_b1, dxs1, dxt1)

        for cb in range(NCB):
            c0 = 16 * cb
            edge = cb == 0 or cb == NCB - 1
            mf = mask_first if cb == 0 else (mask_last if cb == NCB - 1 else None)

            s_a = s_halo[pl.ds(c0, 16)]
            t_a = t_halo[pl.ds(c0, 16)]
            if cb < NCB - 1:
                s_b = s_halo[pl.ds(c0 + 1, 16)]
                t_b = t_halo[pl.ds(c0 + 1, 16)]
            else:
                s_b = s_a.at[perm].get(mode="promise_in_bounds")
                t_b = t_a.at[perm].get(mode="promise_in_bounds")
            prev0 = (s_a, s_b, t_a, t_b, s_b - s_a, t_b - t_a)

            def row_body(g, carry, cb=cb, mf=mf, edge=edge):
                prev = carry[:6]
                accs = list(carry[6:])
                for u in range(RU):
                    cur = loads(RU * g + u, cb)
                    term, prev = term_of(prev, cur)
                    if edge:
                        accs[u] = accs[u] - mf * term
                    else:
                        accs[u] = accs[u] - term
                return prev + tuple(accs)

            zero16 = jnp.zeros((16,), jnp.float32)
            carry = prev0 + (acc, zero16, zero16, zero16)
            carry = plsc.parallel_loop(0, CR // RU, unroll=2, carry=carry)(row_body)
            acc = (carry[6] + carry[7]) + (carry[8] + carry[9])

            e_prev = loads(edge_k - 1, cb)
            e_prev = e_prev + (e_prev[1] - e_prev[0], e_prev[3] - e_prev[2])
            e_term, _ = term_of(e_prev, loads(edge_k, cb))
            if edge:
                acc = acc + ef * (mf * e_term)
            else:
                acc = acc + ef * e_term
        return acc

    acc = lax.fori_loop(0, NCHUNK, chunk_body, jnp.zeros((16,), jnp.float32))

    acc_v[...] = acc
    pltpu.sync_copy(acc_v, out_hbm.at[wid])


def _reduce_body(p_ref, o_ref):
    o_ref[0, 0] = jnp.sum(p_ref[...]) * SCALE


_reduce = pl.pallas_call(
    _reduce_body,
    out_shape=jax.ShapeDtypeStruct((1, 1), jnp.float32),
    out_specs=pl.BlockSpec(memory_space=pltpu.SMEM),
)


def kernel(pred):
    partials = _det_partials(pred)
    return _reduce(partials)[0, 0]


# speedup vs baseline: 413.8649x; 1.2256x over previous
import functools

import jax
import jax.numpy as jnp
from jax import lax
from jax.experimental import pallas as pl
from jax.experimental.pallas import tpu as pltpu
from jax.experimental.pallas import tpu_sc as plsc

H = 384
W = 384
N = 16
NW = 32
CR = 48
NB = CR // 8
NCHUNK = 4
NCB = W // 16
NTC = W // 128
F = (H - 1) * (W - 1) * 2
SCALE = 1.0 / (N * F)

_mesh = plsc.VectorSubcoreMesh(core_axis_name="c", subcore_axis_name="s")


@functools.partial(
    pl.kernel,
    out_type=jax.ShapeDtypeStruct((NW, 16), jnp.float32),
    mesh=_mesh,
    scratch_types=[
        pltpu.VMEM((2, NB, NTC, 8, 128), jnp.float32),
        pltpu.VMEM((2, NB, NTC, 8, 128), jnp.float32),
        pltpu.VMEM((W,), jnp.float32),
        pltpu.VMEM((W,), jnp.float32),
        pltpu.VMEM((NTC, 8, 128), jnp.float32),
        pltpu.VMEM((16,), jnp.float32),
        pltpu.SemaphoreType.DMA((2,)),
        pltpu.SemaphoreType.DMA((2,)),
    ],
)
def _det_partials(pred_hbm, out_hbm, s_sl, t_sl, s_halo, t_halo, stage, acc_v,
                  sem_s, sem_t):
    wid = lax.axis_index("s") * 2 + lax.axis_index("c")
    n = wid // 2
    h = wid % 2

    lane = lax.iota(jnp.int32, 16)
    mask_first = jnp.where(lane >= 1, 1.0, 0.0).astype(jnp.float32)
    mask_last = jnp.where(lane < 14, 1.0, 0.0).astype(jnp.float32)
    perm = jnp.minimum(lane + 1, 15)
    zidx = lane - lane

    edge_j = jnp.where(h == 0, 0, NCHUNK - 1)
    e_bb = jnp.where(h == 0, 0, NB - 1)
    e_r8a = jnp.where(h == 0, 0, 6)
    e_r8b = jnp.where(h == 0, 1, 7)

    def win_src(ch_idx, j):
        return pred_hbm.at[n, ch_idx, pl.ds(h * 24 + j * 6, NB)]

    pltpu.async_copy(win_src(0, 0), s_sl.at[0], sem_s.at[0])
    pltpu.async_copy(win_src(1, 0), t_sl.at[0], sem_t.at[0])

    @pl.when(h == 1)
    def _():
        pltpu.sync_copy(pred_hbm.at[n, 0, 23], stage)
        for i in range(NCB):
            s_halo[pl.ds(16 * i, 16)] = stage[i // 8, 7, pl.ds((16 * i) % 128, 16)]
        pltpu.sync_copy(pred_hbm.at[n, 1, 23], stage)
        for i in range(NCB):
            t_halo[pl.ds(16 * i, 16)] = stage[i // 8, 7, pl.ds((16 * i) % 128, 16)]

    def chunk_body(j, acc):
        slot = j % 2
        other = 1 - slot

        pltpu.make_async_copy(win_src(0, j), s_sl.at[slot], sem_s.at[slot]).wait()
        pltpu.make_async_copy(win_src(1, j), t_sl.at[slot], sem_t.at[slot]).wait()

        @pl.when(jnp.logical_and(j == 0, h == 0))
        def _():
            for i in range(NCB):
                s_halo[pl.ds(16 * i, 16)] = s_sl[slot, 0, i // 8, 0,
                                                 pl.ds((16 * i) % 128, 16)]
                t_halo[pl.ds(16 * i, 16)] = t_sl[slot, 0, i // 8, 0,
                                                 pl.ds((16 * i) % 128, 16)]

        @pl.when(j > 0)
        def _():
            for i in range(NCB):
                s_halo[pl.ds(16 * i, 16)] = s_sl[other, NB - 1, i // 8, 7,
                                                 pl.ds((16 * i) % 128, 16)]
                t_halo[pl.ds(16 * i, 16)] = t_sl[other, NB - 1, i // 8, 7,
                                                 pl.ds((16 * i) % 128, 16)]

        @pl.when(j < NCHUNK - 1)
        def _():
            pltpu.async_copy(win_src(0, j + 1), s_sl.at[other], sem_s.at[other])
            pltpu.async_copy(win_src(1, j + 1), t_sl.at[other], sem_t.at[other])

        ef = jnp.where(j == edge_j, 1.0, 0.0).astype(jnp.float32)

        def loads(bb, r8, cb):
            tc, cw = cb // 8, (cb % 8) * 16
            s_a1 = s_sl[slot, bb, tc, r8, pl.ds(cw, 16)]
            t_a1 = t_sl[slot, bb, tc, r8, pl.ds(cw, 16)]
            if cb % 8 < 7:
                s_b1 = s_sl[slot, bb, tc, r8, pl.ds(cw + 1, 16)]
                t_b1 = t_sl[slot, bb, tc, r8, pl.ds(cw + 1, 16)]
            elif cb < NCB - 1:
                s_n = s_sl[slot, bb, tc + 1, r8, pl.ds(0, 16)]
                t_n = t_sl[slot, bb, tc + 1, r8, pl.ds(0, 16)]
                s_b1 = jnp.where(lane < 15,
                                 s_a1.at[perm].get(mode="promise_in_bounds"),
                                 s_n.at[zidx].get(mode="promise_in_bounds"))
                t_b1 = jnp.where(lane < 15,
                                 t_a1.at[perm].get(mode="promise_in_bounds"),
                                 t_n.at[zidx].get(mode="promise_in_bounds"))
            else:
                s_b1 = s_a1.at[perm].get(mode="promise_in_bounds")
                t_b1 = t_a1.at[perm].get(mode="promise_in_bounds")
            return s_a1, s_b1, t_a1, t_b1

        def term_of(prev, cur):
            s_a, s_b, t_a, t_b, dxs0, dxt0 = prev
            s_a1, s_b1, t_a1, t_b1 = cur
            dys = s_a1 - s_a
            dyt = t_a1 - t_a
            dxs1 = s_b1 - s_a1
            dxt1 = t_b1 - t_a1
            det1 = dxs0 * dyt - dxt0 * dys
            dys_b = s_b1 - s_b
            dyt_b = t_b1 - t_b
            det2 = dxs1 * dyt_b - dxt1 * dys_b
            term = jnp.minimum(det1, 0.0) + jnp.minimum(det2, 0.0)
            return term, (s_a1, s_b1, t_a1, t_b1, dxs1, dxt1)

        for cb in range(NCB):
            c0 = 16 * cb
            edge = cb == 0 or cb == NCB - 1
            mf = mask_first if cb == 0 else (mask_last if cb == NCB - 1 else None)

            s_a = s_halo[pl.ds(c0, 16)]
            t_a = t_halo[pl.ds(c0, 16)]
            if cb < NCB - 1:
                s_b = s_halo[pl.ds(c0 + 1, 16)]
                t_b = t_halo[pl.ds(c0 + 1, 16)]
            else:
                s_b = s_a.at[perm].get(mode="promise_in_bounds")
                t_b = t_a.at[perm].get(mode="promise_in_bounds")
            prev0 = (s_a, s_b, t_a, t_b, s_b - s_a, t_b - t_a)

            def band_body(bb, carry, cb=cb, mf=mf, edge=edge):
                prev = carry[:6]
                accs = list(carry[6:])
                for r8 in range(8):
                    term, prev = term_of(prev, loads(bb, r8, cb))
                    if edge:
                        accs[r8 % 4] = accs[r8 % 4] - mf * term
                    else:
                        accs[r8 % 4] = accs[r8 % 4] - term
                return prev + tuple(accs)

            zero16 = jnp.zeros((16,), jnp.float32)
            carry = prev0 + (acc, zero16, zero16, zero16)
            carry = plsc.parallel_loop(0, NB, unroll=1, carry=carry)(band_body)
            acc = (carry[6] + carry[7]) + (carry[8] + carry[9])

            e_prev = loads(e_bb, e_r8a, cb)
            e_prev = e_prev + (e_prev[1] - e_prev[0], e_prev[3] - e_prev[2])
            e_term, _ = term_of(e_prev, loads(e_bb, e_r8b, cb))
            if edge:
                acc = acc + ef * (mf * e_term)
            else:
                acc = acc + ef * e_term
        return acc

    acc = lax.fori_loop(0, NCHUNK, chunk_body, jnp.zeros((16,), jnp.float32))

    acc_v[...] = acc
    pltpu.sync_copy(acc_v, out_hbm.at[wid])


def _reduce_body(p_ref, o_ref):
    o_ref[0, 0] = jnp.sum(p_ref[...]) * SCALE


_reduce = pl.pallas_call(
    _reduce_body,
    out_shape=jax.ShapeDtypeStruct((1, 1), jnp.float32),
    out_specs=pl.BlockSpec(memory_space=pltpu.SMEM),
)


def kernel(pred):
    pred_t = pred.reshape(N, 2, H // 8, 8, W // 128, 128).transpose(0, 1, 2, 4, 3, 5)
    partials = _det_partials(pred_t)
    return _reduce(partials)[0, 0]


# speedup vs baseline: 537.3901x; 1.2985x over previous
import functools

import jax
import jax.numpy as jnp
from jax import lax
from jax.experimental import pallas as pl
from jax.experimental.pallas import tpu as pltpu
from jax.experimental.pallas import tpu_sc as plsc

H = 384
W = 384
N = 16
NW = 32
CR = 48
NB = CR // 8
NCHUNK = 2
NSC = 8
NCB = W // 16
NTC = W // 128
F = (H - 1) * (W - 1) * 2
SCALE = 1.0 / (N * F)

_mesh = plsc.VectorSubcoreMesh(core_axis_name="c", subcore_axis_name="s")


@functools.partial(
    pl.kernel,
    out_type=jax.ShapeDtypeStruct((NW, 16), jnp.float32),
    mesh=_mesh,
    scratch_types=[
        pltpu.VMEM((2, NB, NTC, 8, 128), jnp.float32),
        pltpu.VMEM((2, NB, NTC, 8, 128), jnp.float32),
        pltpu.VMEM((W,), jnp.float32),
        pltpu.VMEM((W,), jnp.float32),
        pltpu.VMEM((NTC, 8, 128), jnp.float32),
        pltpu.VMEM((16,), jnp.float32),
        pltpu.SemaphoreType.DMA((2,)),
        pltpu.SemaphoreType.DMA((2,)),
    ],
)
def _det_partials(pred_hbm, out_hbm, s_sl, t_sl, s_halo, t_halo, stage, acc_v,
                  sem_s, sem_t):
    wid = lax.axis_index("s") * 2 + lax.axis_index("c")
    n = wid // 4
    q = wid % 4

    lane = lax.iota(jnp.int32, 16)
    mask_first = jnp.where(lane >= 1, 1.0, 0.0).astype(jnp.float32)
    mask_last = jnp.where(lane < 14, 1.0, 0.0).astype(jnp.float32)
    perm = jnp.minimum(lane + 1, 15)
    zidx = lane - lane

    edge_j = jnp.where(q == 0, 0, jnp.where(q == 3, NCHUNK - 1, -1))
    e_bb = jnp.where(q == 0, 0, NB - 1)
    e_r8a = jnp.where(q == 0, 0, 6)
    e_r8b = jnp.where(q == 0, 1, 7)

    def win_src(ch_idx, j):
        return pred_hbm.at[n, ch_idx, pl.ds(q * 12 + j * 6, NB)]

    pltpu.async_copy(win_src(0, 0), s_sl.at[0], sem_s.at[0])
    pltpu.async_copy(win_src(1, 0), t_sl.at[0], sem_t.at[0])

    @pl.when(q > 0)
    def _():
        pltpu.sync_copy(pred_hbm.at[n, 0, q * 12 - 1], stage)
        for i in range(NCB):
            s_halo[pl.ds(16 * i, 16)] = stage[i // 8, 7, pl.ds((16 * i) % 128, 16)]
        pltpu.sync_copy(pred_hbm.at[n, 1, q * 12 - 1], stage)
        for i in range(NCB):
            t_halo[pl.ds(16 * i, 16)] = stage[i // 8, 7, pl.ds((16 * i) % 128, 16)]

    def chunk_body(j, acc):
        slot = j % 2
        other = 1 - slot

        pltpu.make_async_copy(win_src(0, j), s_sl.at[slot], sem_s.at[slot]).wait()
        pltpu.make_async_copy(win_src(1, j), t_sl.at[slot], sem_t.at[slot]).wait()

        @pl.when(jnp.logical_and(j == 0, q == 0))
        def _():
            for i in range(NCB):
                s_halo[pl.ds(16 * i, 16)] = s_sl[slot, 0, i // 8, 0,
                                                 pl.ds((16 * i) % 128, 16)]
                t_halo[pl.ds(16 * i, 16)] = t_sl[slot, 0, i // 8, 0,
                                                 pl.ds((16 * i) % 128, 16)]

        @pl.when(j > 0)
        def _():
            for i in range(NCB):
                s_halo[pl.ds(16 * i, 16)] = s_sl[other, NB - 1, i // 8, 7,
                                                 pl.ds((16 * i) % 128, 16)]
                t_halo[pl.ds(16 * i, 16)] = t_sl[other, NB - 1, i // 8, 7,
                                                 pl.ds((16 * i) % 128, 16)]

        @pl.when(j < NCHUNK - 1)
        def _():
            pltpu.async_copy(win_src(0, j + 1), s_sl.at[other], sem_s.at[other])
            pltpu.async_copy(win_src(1, j + 1), t_sl.at[other], sem_t.at[other])

        ef = jnp.where(j == edge_j, 1.0, 0.0).astype(jnp.float32)

        def loads(bb, r8, cb):
            tc, cw = cb // 8, (cb % 8) * 16
            s_a1 = s_sl[slot, bb, tc, r8, pl.ds(cw, 16)]
            t_a1 = t_sl[slot, bb, tc, r8, pl.ds(cw, 16)]
            if cb % 8 < 7:
                s_b1 = s_sl[slot, bb, tc, r8, pl.ds(cw + 1, 16)]
                t_b1 = t_sl[slot, bb, tc, r8, pl.ds(cw + 1, 16)]
            elif cb < NCB - 1:
                s_n = s_sl[slot, bb, tc + 1, r8, pl.ds(0, 16)]
                t_n = t_sl[slot, bb, tc + 1, r8, pl.ds(0, 16)]
                s_b1 = jnp.where(lane < 15,
                                 s_a1.at[perm].get(mode="promise_in_bounds"),
                                 s_n.at[zidx].get(mode="promise_in_bounds"))
                t_b1 = jnp.where(lane < 15,
                                 t_a1.at[perm].get(mode="promise_in_bounds"),
                                 t_n.at[zidx].get(mode="promise_in_bounds"))
            else:
                s_b1 = s_a1.at[perm].get(mode="promise_in_bounds")
                t_b1 = t_a1.at[perm].get(mode="promise_in_bounds")
            return s_a1, s_b1, t_a1, t_b1

        def term_of(prev, cur):
            s_a, s_b, t_a, t_b, dxs0, dxt0 = prev
            s_a1, s_b1, t_a1, t_b1 = cur
            dys = s_a1 - s_a
            dyt = t_a1 - t_a
            dxs1 = s_b1 - s_a1
            dxt1 = t_b1 - t_a1
            det1 = dxs0 * dyt - dxt0 * dys
            dys_b = s_b1 - s_b
            dyt_b = t_b1 - t_b
            det2 = dxs1 * dyt_b - dxt1 * dys_b
            term = jnp.minimum(det1, 0.0) + jnp.minimum(det2, 0.0)
            return term, (s_a1, s_b1, t_a1, t_b1, dxs1, dxt1)

        for cb in range(NCB):
            c0 = 16 * cb
            edge = cb == 0 or cb == NCB - 1
            mf = mask_first if cb == 0 else (mask_last if cb == NCB - 1 else None)

            s_a = s_halo[pl.ds(c0, 16)]
            t_a = t_halo[pl.ds(c0, 16)]
            if cb < NCB - 1:
                s_b = s_halo[pl.ds(c0 + 1, 16)]
                t_b = t_halo[pl.ds(c0 + 1, 16)]
            else:
                s_b = s_a.at[perm].get(mode="promise_in_bounds")
                t_b = t_a.at[perm].get(mode="promise_in_bounds")
            prev0 = (s_a, s_b, t_a, t_b, s_b - s_a, t_b - t_a)

            def band_body(bb, carry, cb=cb, mf=mf, edge=edge):
                prev = carry[:6]
                accs = list(carry[6:])
                for r8 in range(8):
                    term, prev = term_of(prev, loads(bb, r8, cb))
                    if edge:
                        accs[r8 % 4] = accs[r8 % 4] - mf * term
                    else:
                        accs[r8 % 4] = accs[r8 % 4] - term
                return prev + tuple(accs)

            zero16 = jnp.zeros((16,), jnp.float32)
            carry = prev0 + (acc, zero16, zero16, zero16)
            carry = plsc.parallel_loop(0, NB, unroll=1, carry=carry)(band_body)
            acc = (carry[6] + carry[7]) + (carry[8] + carry[9])

            e_prev = loads(e_bb, e_r8a, cb)
            e_prev = e_prev + (e_prev[1] - e_prev[0], e_prev[3] - e_prev[2])
            e_term, _ = term_of(e_prev, loads(e_bb, e_r8b, cb))
            if edge:
                acc = acc + ef * (mf * e_term)
            else:
                acc = acc + ef * e_term
        return acc

    acc = lax.fori_loop(0, NCHUNK, chunk_body, jnp.zeros((16,), jnp.float32))

    acc_v[...] = acc
    pltpu.sync_copy(acc_v, out_hbm.at[wid])


def _tc_body(p_ref, o_ref):
    s = p_ref[0, 0]
    t = p_ref[0, 1]
    s_r = pltpu.roll(s, H - 1, 0)
    t_r = pltpu.roll(t, H - 1, 0)
    s_c = pltpu.roll(s, W - 1, 1)
    t_c = pltpu.roll(t, W - 1, 1)
    s_rc = pltpu.roll(s_c, H - 1, 0)
    t_rc = pltpu.roll(t_c, H - 1, 0)
    det1 = (s_c - s) * (t_r - t) - (t_c - t) * (s_r - s)
    det2 = (s_rc - s_r) * (t_rc - t_c) - (t_rc - t_r) * (s_rc - s_c)
    rows = lax.broadcasted_iota(jnp.int32, (H, W), 0)
    cols = lax.broadcasted_iota(jnp.int32, (H, W), 1)
    interior = ((rows >= 1) & (rows <= H - 3) & (cols >= 1) & (cols <= W - 3))
    term = jnp.minimum(det1, 0.0) + jnp.minimum(det2, 0.0)
    o_ref[pl.program_id(0), 0] = -jnp.sum(jnp.where(interior, term, 0.0))


_tc_part = pl.pallas_call(
    _tc_body,
    grid=(N - NSC,),
    in_specs=[pl.BlockSpec((1, 2, H, W), lambda b: (b + NSC, 0, 0, 0))],
    out_specs=pl.BlockSpec((N - NSC, 1), lambda b: (0, 0), memory_space=pltpu.SMEM),
    out_shape=jax.ShapeDtypeStruct((N - NSC, 1), jnp.float32),
)


def _reduce_body(p_ref, q_ref, o_ref):
    o_ref[0, 0] = (jnp.sum(p_ref[...]) + jnp.sum(q_ref[...])) * SCALE


_reduce = pl.pallas_call(
    _reduce_body,
    out_shape=jax.ShapeDtypeStruct((1, 1), jnp.float32),
    out_specs=pl.BlockSpec(memory_space=pltpu.SMEM),
)


def kernel(pred):
    pred_t = pred.reshape(N, 2, H // 8, 8, W // 128, 128).transpose(0, 1, 2, 4, 3, 5)
    partials = _det_partials(pred_t)
    tc_partials = _tc_part(pred)
    return _reduce(partials, tc_partials)[0, 0]


# speedup vs baseline: 538.3821x; 1.0018x over previous
import functools

import jax
import jax.numpy as jnp
from jax import lax
from jax.experimental import pallas as pl
from jax.experimental.pallas import tpu as pltpu
from jax.experimental.pallas import tpu_sc as plsc

H = 384
W = 384
N = 16
NW = 32
CR = 48
NB = CR // 8
NCHUNK = 2
NSC = 8
NCB = W // 16
NTC = W // 128
F = (H - 1) * (W - 1) * 2
SCALE = 1.0 / (N * F)

_mesh = plsc.VectorSubcoreMesh(core_axis_name="c", subcore_axis_name="s")


@functools.partial(
    pl.kernel,
    out_type=jax.ShapeDtypeStruct((NW, 16), jnp.float32),
    mesh=_mesh,
    scratch_types=[
        pltpu.VMEM((2, NB, NTC, 8, 128), jnp.float32),
        pltpu.VMEM((2, NB, NTC, 8, 128), jnp.float32),
        pltpu.VMEM((W,), jnp.float32),
        pltpu.VMEM((W,), jnp.float32),
        pltpu.VMEM((NTC, 8, 128), jnp.float32),
        pltpu.VMEM((16,), jnp.float32),
        pltpu.SemaphoreType.DMA((2,)),
        pltpu.SemaphoreType.DMA((2,)),
    ],
)
def _det_partials(pred_hbm, out_hbm, s_sl, t_sl, s_halo, t_halo, stage, acc_v,
                  sem_s, sem_t):
    wid = lax.axis_index("s") * 2 + lax.axis_index("c")
    n = wid // 4
    q = wid % 4

    lane = lax.iota(jnp.int32, 16)
    mask_first = jnp.where(lane >= 1, 1.0, 0.0).astype(jnp.float32)
    mask_last = jnp.where(lane < 14, 1.0, 0.0).astype(jnp.float32)
    perm = jnp.minimum(lane + 1, 15)
    zidx = lane - lane

    edge_j = jnp.where(q == 0, 0, jnp.where(q == 3, NCHUNK - 1, -1))
    e_bb = jnp.where(q == 0, 0, NB - 1)
    e_r8a = jnp.where(q == 0, 0, 6)
    e_r8b = jnp.where(q == 0, 1, 7)

    def win_src(ch_idx, j):
        return pred_hbm.at[n, ch_idx, pl.ds(q * 12 + j * 6, NB)]

    pltpu.async_copy(win_src(0, 0), s_sl.at[0], sem_s.at[0])
    pltpu.async_copy(win_src(1, 0), t_sl.at[0], sem_t.at[0])

    @pl.when(q > 0)
    def _():
        pltpu.sync_copy(pred_hbm.at[n, 0, q * 12 - 1], stage)
        for i in range(NCB):
            s_halo[pl.ds(16 * i, 16)] = stage[i // 8, 7, pl.ds((16 * i) % 128, 16)]
        pltpu.sync_copy(pred_hbm.at[n, 1, q * 12 - 1], stage)
        for i in range(NCB):
            t_halo[pl.ds(16 * i, 16)] = stage[i // 8, 7, pl.ds((16 * i) % 128, 16)]

    def chunk_body(j, acc):
        slot = j % 2
        other = 1 - slot

        pltpu.make_async_copy(win_src(0, j), s_sl.at[slot], sem_s.at[slot]).wait()
        pltpu.make_async_copy(win_src(1, j), t_sl.at[slot], sem_t.at[slot]).wait()

        @pl.when(jnp.logical_and(j == 0, q == 0))
        def _():
            for i in range(NCB):
                s_halo[pl.ds(16 * i, 16)] = s_sl[slot, 0, i // 8, 0,
                                                 pl.ds((16 * i) % 128, 16)]
                t_halo[pl.ds(16 * i, 16)] = t_sl[slot, 0, i // 8, 0,
                                                 pl.ds((16 * i) % 128, 16)]

        @pl.when(j > 0)
        def _():
            for i in range(NCB):
                s_halo[pl.ds(16 * i, 16)] = s_sl[other, NB - 1, i // 8, 7,
                                                 pl.ds((16 * i) % 128, 16)]
                t_halo[pl.ds(16 * i, 16)] = t_sl[other, NB - 1, i // 8, 7,
                                                 pl.ds((16 * i) % 128, 16)]

        @pl.when(j < NCHUNK - 1)
        def _():
            pltpu.async_copy(win_src(0, j + 1), s_sl.at[other], sem_s.at[other])
            pltpu.async_copy(win_src(1, j + 1), t_sl.at[other], sem_t.at[other])

        ef = jnp.where(j == edge_j, 1.0, 0.0).astype(jnp.float32)

        def loads(bb, r8, cb):
            tc, cw = cb // 8, (cb % 8) * 16
            s_a1 = s_sl[slot, bb, tc, r8, pl.ds(cw, 16)]
            t_a1 = t_sl[slot, bb, tc, r8, pl.ds(cw, 16)]
            if cb % 8 < 7:
                s_b1 = s_sl[slot, bb, tc, r8, pl.ds(cw + 1, 16)]
                t_b1 = t_sl[slot, bb, tc, r8, pl.ds(cw + 1, 16)]
            elif cb < NCB - 1:
                s_n = s_sl[slot, bb, tc + 1, r8, pl.ds(0, 16)]
                t_n = t_sl[slot, bb, tc + 1, r8, pl.ds(0, 16)]
                s_b1 = jnp.where(lane < 15,
                                 s_a1.at[perm].get(mode="promise_in_bounds"),
                                 s_n.at[zidx].get(mode="promise_in_bounds"))
                t_b1 = jnp.where(lane < 15,
                                 t_a1.at[perm].get(mode="promise_in_bounds"),
                                 t_n.at[zidx].get(mode="promise_in_bounds"))
            else:
                s_b1 = s_a1.at[perm].get(mode="promise_in_bounds")
                t_b1 = t_a1.at[perm].get(mode="promise_in_bounds")
            return s_a1, s_b1, t_a1, t_b1

        def term_of(prev, cur):
            s_a, s_b, t_a, t_b, dxs0, dxt0 = prev
            s_a1, s_b1, t_a1, t_b1 = cur
            dys = s_a1 - s_a
            dyt = t_a1 - t_a
            dxs1 = s_b1 - s_a1
            dxt1 = t_b1 - t_a1
            det1 = dxs0 * dyt - dxt0 * dys
            dys_b = s_b1 - s_b
            dyt_b = t_b1 - t_b
            det2 = dxs1 * dyt_b - dxt1 * dys_b
            term = jnp.minimum(det1, 0.0) + jnp.minimum(det2, 0.0)
            return term, (s_a1, s_b1, t_a1, t_b1, dxs1, dxt1)

        for cb in range(NCB):
            c0 = 16 * cb
            edge = cb == 0 or cb == NCB - 1
            mf = mask_first if cb == 0 else (mask_last if cb == NCB - 1 else None)

            s_a = s_halo[pl.ds(c0, 16)]
            t_a = t_halo[pl.ds(c0, 16)]
            if cb < NCB - 1:
                s_b = s_halo[pl.ds(c0 + 1, 16)]
                t_b = t_halo[pl.ds(c0 + 1, 16)]
            else:
                s_b = s_a.at[perm].get(mode="promise_in_bounds")
                t_b = t_a.at[perm].get(mode="promise_in_bounds")
            prev0 = (s_a, s_b, t_a, t_b, s_b - s_a, t_b - t_a)

            def band_body(bb, carry, cb=cb, mf=mf, edge=edge):
                prev = carry[:6]
                accs = list(carry[6:])
                for r8 in range(8):
                    term, prev = term_of(prev, loads(bb, r8, cb))
                    if edge:
                        accs[r8 % 4] = accs[r8 % 4] - mf * term
                    else:
                        accs[r8 % 4] = accs[r8 % 4] - term
                return prev + tuple(accs)

            zero16 = jnp.zeros((16,), jnp.float32)
            carry = prev0 + (acc, zero16, zero16, zero16)
            carry = plsc.parallel_loop(0, NB, unroll=2, carry=carry)(band_body)
            acc = (carry[6] + carry[7]) + (carry[8] + carry[9])

            e_prev = loads(e_bb, e_r8a, cb)
            e_prev = e_prev + (e_prev[1] - e_prev[0], e_prev[3] - e_prev[2])
            e_term, _ = term_of(e_prev, loads(e_bb, e_r8b, cb))
            if edge:
                acc = acc + ef * (mf * e_term)
            else:
                acc = acc + ef * e_term
        return acc

    acc = lax.fori_loop(0, NCHUNK, chunk_body, jnp.zeros((16,), jnp.float32))

    acc_v[...] = acc
    pltpu.sync_copy(acc_v, out_hbm.at[wid])


def _tc_body(p_ref, o_ref):
    s = p_ref[0, 0]
    t = p_ref[0, 1]
    s_r = pltpu.roll(s, H - 1, 0)
    t_r = pltpu.roll(t, H - 1, 0)
    s_c = pltpu.roll(s, W - 1, 1)
    t_c = pltpu.roll(t, W - 1, 1)
    s_rc = pltpu.roll(s_c, H - 1, 0)
    t_rc = pltpu.roll(t_c, H - 1, 0)
    det1 = (s_c - s) * (t_r - t) - (t_c - t) * (s_r - s)
    det2 = (s_rc - s_r) * (t_rc - t_c) - (t_rc - t_r) * (s_rc - s_c)
    rows = lax.broadcasted_iota(jnp.int32, (H, W), 0)
    cols = lax.broadcasted_iota(jnp.int32, (H, W), 1)
    interior = ((rows >= 1) & (rows <= H - 3) & (cols >= 1) & (cols <= W - 3))
    term = jnp.minimum(det1, 0.0) + jnp.minimum(det2, 0.0)
    o_ref[pl.program_id(0), 0] = -jnp.sum(jnp.where(interior, term, 0.0))


_tc_part = pl.pallas_call(
    _tc_body,
    grid=(N - NSC,),
    in_specs=[pl.BlockSpec((1, 2, H, W), lambda b: (b + NSC, 0, 0, 0))],
    out_specs=pl.BlockSpec((N - NSC, 1), lambda b: (0, 0), memory_space=pltpu.SMEM),
    out_shape=jax.ShapeDtypeStruct((N - NSC, 1), jnp.float32),
)


def _reduce_body(p_ref, q_ref, o_ref):
    o_ref[0, 0] = (jnp.sum(p_ref[...]) + jnp.sum(q_ref[...])) * SCALE


_reduce = pl.pallas_call(
    _reduce_body,
    out_shape=jax.ShapeDtypeStruct((1, 1), jnp.float32),
    out_specs=pl.BlockSpec(memory_space=pltpu.SMEM),
)


def kernel(pred):
    pred_t = pred.reshape(N, 2, H // 8, 8, W // 128, 128).transpose(0, 1, 2, 4, 3, 5)
    partials = _det_partials(pred_t)
    tc_partials = _tc_part(pred)
    return _reduce(partials, tc_partials)[0, 0]


# speedup vs baseline: 538.9724x; 1.0011x over previous
import functools

import jax
import jax.numpy as jnp
from jax import lax
from jax.experimental import pallas as pl
from jax.experimental.pallas import tpu as pltpu
from jax.experimental.pallas import tpu_sc as plsc

H = 384
W = 384
N = 16
NW = 32
CR = 48
NB = CR // 8
NCHUNK = 2
NSC = 8
NCB = W // 16
NTC = W // 128
F = (H - 1) * (W - 1) * 2
SCALE = 1.0 / (N * F)

_mesh = plsc.VectorSubcoreMesh(core_axis_name="c", subcore_axis_name="s")


@functools.partial(
    pl.kernel,
    out_type=jax.ShapeDtypeStruct((NW, 16), jnp.float32),
    mesh=_mesh,
    scratch_types=[
        pltpu.VMEM((2, NB, NTC, 8, 128), jnp.float32),
        pltpu.VMEM((2, NB, NTC, 8, 128), jnp.float32),
        pltpu.VMEM((W,), jnp.float32),
        pltpu.VMEM((W,), jnp.float32),
        pltpu.VMEM((NTC, 8, 128), jnp.float32),
        pltpu.VMEM((16,), jnp.float32),
        pltpu.SemaphoreType.DMA((2,)),
        pltpu.SemaphoreType.DMA((2,)),
    ],
)
def _det_partials(pred_hbm, out_hbm, s_sl, t_sl, s_halo, t_halo, stage, acc_v,
                  sem_s, sem_t):
    wid = lax.axis_index("s") * 2 + lax.axis_index("c")
    n = wid // 4
    q = wid % 4

    lane = lax.iota(jnp.int32, 16)
    mask_first = jnp.where(lane >= 1, 1.0, 0.0).astype(jnp.float32)
    mask_last = jnp.where(lane < 14, 1.0, 0.0).astype(jnp.float32)
    perm = jnp.minimum(lane + 1, 15)
    zidx = lane - lane

    edge_j = jnp.where(q == 0, 0, jnp.where(q == 3, NCHUNK - 1, -1))
    e_bb = jnp.where(q == 0, 0, NB - 1)
    e_r8a = jnp.where(q == 0, 0, 6)
    e_r8b = jnp.where(q == 0, 1, 7)

    def win_src(ch_idx, j):
        return pred_hbm.at[n, ch_idx, pl.ds(q * 12 + j * 6, NB)]

    pltpu.async_copy(win_src(0, 0), s_sl.at[0], sem_s.at[0])
    pltpu.async_copy(win_src(1, 0), t_sl.at[0], sem_t.at[0])

    @pl.when(q > 0)
    def _():
        pltpu.sync_copy(pred_hbm.at[n, 0, q * 12 - 1], stage)
        for i in range(NCB):
            s_halo[pl.ds(16 * i, 16)] = stage[i // 8, 7, pl.ds((16 * i) % 128, 16)]
        pltpu.sync_copy(pred_hbm.at[n, 1, q * 12 - 1], stage)
        for i in range(NCB):
            t_halo[pl.ds(16 * i, 16)] = stage[i // 8, 7, pl.ds((16 * i) % 128, 16)]

    def chunk_body(j, acc):
        slot = j % 2
        other = 1 - slot

        pltpu.make_async_copy(win_src(0, j), s_sl.at[slot], sem_s.at[slot]).wait()
        pltpu.make_async_copy(win_src(1, j), t_sl.at[slot], sem_t.at[slot]).wait()

        @pl.when(jnp.logical_and(j == 0, q == 0))
        def _():
            for i in range(NCB):
                s_halo[pl.ds(16 * i, 16)] = s_sl[slot, 0, i // 8, 0,
                                                 pl.ds((16 * i) % 128, 16)]
                t_halo[pl.ds(16 * i, 16)] = t_sl[slot, 0, i // 8, 0,
                                                 pl.ds((16 * i) % 128, 16)]

        @pl.when(j > 0)
        def _():
            for i in range(NCB):
                s_halo[pl.ds(16 * i, 16)] = s_sl[other, NB - 1, i // 8, 7,
                                                 pl.ds((16 * i) % 128, 16)]
                t_halo[pl.ds(16 * i, 16)] = t_sl[other, NB - 1, i // 8, 7,
                                                 pl.ds((16 * i) % 128, 16)]

        @pl.when(j < NCHUNK - 1)
        def _():
            pltpu.async_copy(win_src(0, j + 1), s_sl.at[other], sem_s.at[other])
            pltpu.async_copy(win_src(1, j + 1), t_sl.at[other], sem_t.at[other])

        ef = jnp.where(j == edge_j, 1.0, 0.0).astype(jnp.float32)

        def loads(bb, r8, cb):
            tc, cw = cb // 8, (cb % 8) * 16
            s_a1 = s_sl[slot, bb, tc, r8, pl.ds(cw, 16)]
            t_a1 = t_sl[slot, bb, tc, r8, pl.ds(cw, 16)]
            if cb % 8 < 7:
                s_b1 = s_sl[slot, bb, tc, r8, pl.ds(cw + 1, 16)]
                t_b1 = t_sl[slot, bb, tc, r8, pl.ds(cw + 1, 16)]
            elif cb < NCB - 1:
                s_n = s_sl[slot, bb, tc + 1, r8, pl.ds(0, 16)]
                t_n = t_sl[slot, bb, tc + 1, r8, pl.ds(0, 16)]
                s_b1 = jnp.where(lane < 15,
                                 s_a1.at[perm].get(mode="promise_in_bounds"),
                                 s_n.at[zidx].get(mode="promise_in_bounds"))
                t_b1 = jnp.where(lane < 15,
                                 t_a1.at[perm].get(mode="promise_in_bounds"),
                                 t_n.at[zidx].get(mode="promise_in_bounds"))
            else:
                s_b1 = s_a1.at[perm].get(mode="promise_in_bounds")
                t_b1 = t_a1.at[perm].get(mode="promise_in_bounds")
            return s_a1, s_b1, t_a1, t_b1

        def term_of(prev, cur):
            s_a, s_b, t_a, t_b, dxs0, dxt0 = prev
            s_a1, s_b1, t_a1, t_b1 = cur
            dys = s_a1 - s_a
            dyt = t_a1 - t_a
            dxs1 = s_b1 - s_a1
            dxt1 = t_b1 - t_a1
            det1 = dxs0 * dyt - dxt0 * dys
            dys_b = s_b1 - s_b
            dyt_b = t_b1 - t_b
            det2 = dxs1 * dyt_b - dxt1 * dys_b
            term = jnp.minimum(det1, 0.0) + jnp.minimum(det2, 0.0)
            return term, (s_a1, s_b1, t_a1, t_b1, dxs1, dxt1)

        for cb in range(NCB):
            c0 = 16 * cb
            edge = cb == 0 or cb == NCB - 1
            mf = mask_first if cb == 0 else (mask_last if cb == NCB - 1 else None)

            s_a = s_halo[pl.ds(c0, 16)]
            t_a = t_halo[pl.ds(c0, 16)]
            if cb < NCB - 1:
                s_b = s_halo[pl.ds(c0 + 1, 16)]
                t_b = t_halo[pl.ds(c0 + 1, 16)]
            else:
                s_b = s_a.at[perm].get(mode="promise_in_bounds")
                t_b = t_a.at[perm].get(mode="promise_in_bounds")
            prev0 = (s_a, s_b, t_a, t_b, s_b - s_a, t_b - t_a)

            def band_body(bb, carry, cb=cb, mf=mf, edge=edge):
                prev = carry[:6]
                accs = list(carry[6:])
                for r8 in range(8):
                    term, prev = term_of(prev, loads(bb, r8, cb))
                    if edge:
                        accs[r8 % 4] = accs[r8 % 4] - mf * term
                    else:
                        accs[r8 % 4] = accs[r8 % 4] - term
                return prev + tuple(accs)

            zero16 = jnp.zeros((16,), jnp.float32)
            carry = prev0 + (acc, zero16, zero16, zero16)
            carry = plsc.parallel_loop(0, NB, unroll=1, carry=carry)(band_body)
            acc = (carry[6] + carry[7]) + (carry[8] + carry[9])

            e_prev = loads(e_bb, e_r8a, cb)
            e_prev = e_prev + (e_prev[1] - e_prev[0], e_prev[3] - e_prev[2])
            e_term, _ = term_of(e_prev, loads(e_bb, e_r8b, cb))
            if edge:
                acc = acc + ef * (mf * e_term)
            else:
                acc = acc + ef * e_term
        return acc

    acc = lax.fori_loop(0, NCHUNK, chunk_body, jnp.zeros((16,), jnp.float32))

    acc_v[...] = acc
    pltpu.sync_copy(acc_v, out_hbm.at[wid])


def _tc_body(p_ref, o_ref):
    s = p_ref[0, 0]
    t = p_ref[0, 1]
    s_r = pltpu.roll(s, H - 1, 0)
    t_r = pltpu.roll(t, H - 1, 0)
    s_c = pltpu.roll(s, W - 1, 1)
    t_c = pltpu.roll(t, W - 1, 1)
    s_rc = pltpu.roll(s_c, H - 1, 0)
    t_rc = pltpu.roll(t_c, H - 1, 0)
    det1 = (s_c - s) * (t_r - t) - (t_c - t) * (s_r - s)
    det2 = (s_rc - s_r) * (t_rc - t_c) - (t_rc - t_r) * (s_rc - s_c)
    rows = lax.broadcasted_iota(jnp.int32, (H, W), 0)
    cols = lax.broadcasted_iota(jnp.int32, (H, W), 1)
    interior = ((rows >= 1) & (rows <= H - 3) & (cols >= 1) & (cols <= W - 3))
    term = jnp.minimum(det1, 0.0) + jnp.minimum(det2, 0.0)
    o_ref[pl.program_id(0), 0] = -jnp.sum(jnp.where(interior, term, 0.0))


_tc_part = pl.pallas_call(
    _tc_body,
    grid=(N - NSC,),
    in_specs=[pl.BlockSpec((1, 2, H, W), lambda b: (b + NSC, 0, 0, 0))],
    out_specs=pl.BlockSpec((N - NSC, 1), lambda b: (0, 0), memory_space=pltpu.SMEM),
    out_shape=jax.ShapeDtypeStruct((N - NSC, 1), jnp.float32),
)


def _reduce_body(p_ref, q_ref, o_ref):
    o_ref[0, 0] = (jnp.sum(p_ref[...]) + jnp.sum(q_ref[...])) * SCALE


_reduce = pl.pallas_call(
    _reduce_body,
    out_shape=jax.ShapeDtypeStruct((1, 1), jnp.float32),
    out_specs=pl.BlockSpec(memory_space=pltpu.SMEM),
)


def kernel(pred):
    pred_t = pred.reshape(N, 2, H // 8, 8, W // 128, 128).transpose(0, 1, 2, 4, 3, 5)
    partials = _det_partials(pred_t)
    tc_partials = _tc_part(pred)
    return _reduce(partials, tc_partials)[0, 0]
